# baseline (speedup 1.0000x reference)
import jax
import jax.numpy as jnp
from jax.experimental import pallas as pl
from jax.experimental.pallas import tpu as pltpu

_ROWS = 128
_COLS = 32768
_C = 8192
_NC = _COLS // _C
_S = 512
_NS = _C // _S


def _scan_kernel(x_ref, u_ref, o_ref, carry_ref):
    i = pl.program_id(0)

    @pl.when(i == 0)
    def _():
        carry_ref[...] = jnp.zeros_like(carry_ref)

    u = u_ref[...]
    p = carry_ref[:, :1]
    for k in range(_NS):
        xk = x_ref[:, k * _S:(k + 1) * _S]
        yk = jax.lax.dot_general(
            xk.astype(jnp.bfloat16), u, (((1,), (0,)), ((), ())),
            preferred_element_type=jnp.float32)
        o_ref[:, k * _S:(k + 1) * _S] = yk + p
        p = p + jnp.sum(xk, axis=1, keepdims=True)
    carry_ref[:, :1] = p


def kernel(x):
    u = jnp.triu(jnp.ones((_S, _S), jnp.bfloat16))
    return pl.pallas_call(
        _scan_kernel,
        grid=(_NC,),
        in_specs=[
            pl.BlockSpec((_ROWS, _C), lambda i: (0, i)),
            pl.BlockSpec((_S, _S), lambda i: (0, 0)),
        ],
        out_specs=pl.BlockSpec((_ROWS, _C), lambda i: (0, i)),
        out_shape=jax.ShapeDtypeStruct((_ROWS, _COLS), jnp.float32),
        scratch_shapes=[pltpu.VMEM((_ROWS, 8), jnp.float32)],
    )(x, u)
